# baseline (speedup 1.0000x reference)
import jax
import jax.numpy as jnp
from jax import lax
from jax.experimental import pallas as pl
from jax.experimental.pallas import tpu as pltpu
from jax.experimental.pallas import tpu_sc as plsc

B = 4096
L = 200
DIM = 64
PAIRS = 500000

NC = 2
NS = 16
NW = NC * NS
BBLK = B // NW
NBUF = 2


def _body(idxT_hbm, w2_hbm, out_hbm, idx_v, jbuf, pairs_v, outb, gsem, osem):
    cid = lax.axis_index("c")
    sid = lax.axis_index("s")
    wid = sid * NC + cid
    b0 = wid * BBLK

    pltpu.sync_copy(idxT_hbm.at[:, pl.ds(b0, BBLK)], idx_v)

    lanes = lax.iota(jnp.int32, 16)

    def fire(r, l):
        for g in range(BBLK // 16):
            v = idx_v[l, pl.ds(g * 16, 16)]
            jbuf[r, pl.ds(g * 16, 16)] = v >> 1
        pltpu.async_copy(w2_hbm.at[jbuf.at[r]], pairs_v.at[r], gsem)

    def drain_store(r, l):
        pltpu.make_async_copy(outb.at[r], out_hbm.at[l, :, pl.ds(b0, BBLK)], osem).wait()

    def consume(r, l):
        pltpu.make_async_copy(w2_hbm.at[jbuf.at[r]], pairs_v.at[r], gsem).wait()
        for g in range(BBLK // 16):
            rows = g * 16 + lanes
            idxg = idx_v[l, pl.ds(g * 16, 16)]
            off = (idxg & 1) * 64

            @pl.loop(0, DIM, unroll=8)
            def _floop(f):
                vec = plsc.load_gather(pairs_v.at[r], [rows, off + f])
                outb[r, f, pl.ds(g * 16, 16)] = vec

        pltpu.async_copy(outb.at[r], out_hbm.at[l, :, pl.ds(b0, BBLK)], osem)

    fire(0, 0)
    fire(1, 1)

    @pl.loop(0, L - NBUF, step=NBUF)
    def _chunk(l0):
        for r in range(NBUF):
            consume(r, l0 + r)
        for r in range(NBUF):
            drain_store(r, l0 + r)
            fire(r, l0 + NBUF + r)

    for r in range(NBUF):
        consume(r, L - NBUF + r)
    for r in range(NBUF):
        drain_store(r, L - NBUF + r)


_mesh = plsc.VectorSubcoreMesh(core_axis_name="c", subcore_axis_name="s")

_gather_call = pl.kernel(
    _body,
    out_type=jax.ShapeDtypeStruct((L, DIM, B), jnp.float32),
    mesh=_mesh,
    scratch_types=[
        pltpu.VMEM((L, BBLK), jnp.int32),
        pltpu.VMEM((NBUF, BBLK), jnp.int32),
        pltpu.VMEM((NBUF, BBLK, 128), jnp.float32),
        pltpu.VMEM((NBUF, DIM, BBLK), jnp.float32),
        pltpu.SemaphoreType.DMA,
        pltpu.SemaphoreType.DMA,
    ],
    compiler_params=pltpu.CompilerParams(needs_layout_passes=False),
)


@jax.jit
def _run(idxT, w2):
    return _gather_call(idxT, w2)


def kernel(gene_indices, weight):
    idxT = jnp.transpose(jnp.asarray(gene_indices, jnp.int32))
    w2 = jnp.reshape(weight, (PAIRS, 128))
    out_t = _run(idxT, w2)
    return jnp.transpose(out_t, (2, 0, 1))


# speedup vs baseline: 1.0084x; 1.0084x over previous
import jax
import jax.numpy as jnp
from jax import lax
from jax.experimental import pallas as pl
from jax.experimental.pallas import tpu as pltpu
from jax.experimental.pallas import tpu_sc as plsc

B = 4096
L = 200
DIM = 64
PAIRS = 500000

NC = 2
NS = 16
NW = NC * NS
BBLK = B // NW
NG = BBLK // 16
NBUF = 2


def _body(idxT_hbm, w2_hbm, out_hbm, idx_v, jbuf, pairs_v, outb, gsem, osem):
    cid = lax.axis_index("c")
    sid = lax.axis_index("s")
    wid = sid * NC + cid
    b0 = wid * BBLK

    pltpu.sync_copy(idxT_hbm.at[:, pl.ds(b0, BBLK)], idx_v)

    lanes = lax.iota(jnp.int32, 16)
    rows = [g * 16 + lanes for g in range(NG)]

    def fire(r, l):
        for g in range(NG):
            v = idx_v[l, pl.ds(g * 16, 16)]
            jbuf[r, pl.ds(g * 16, 16)] = v >> 1
        pltpu.async_copy(w2_hbm.at[jbuf.at[r]], pairs_v.at[r], gsem)

    def drain_store(r, l):
        pltpu.make_async_copy(
            outb.at[r], out_hbm.at[l, :, pl.ds(b0, BBLK)], osem
        ).wait()

    def consume(r, l):
        pltpu.make_async_copy(w2_hbm.at[jbuf.at[r]], pairs_v.at[r], gsem).wait()
        offs = tuple(
            (idx_v[l, pl.ds(g * 16, 16)] & 1) * 64 for g in range(NG)
        )

        @pl.loop(0, DIM, init_carry=offs)
        def _floop(f, cols):
            for g in range(NG):
                vec = plsc.load_gather(pairs_v.at[r], [rows[g], cols[g]])
                outb[r, f, pl.ds(g * 16, 16)] = vec
            return tuple(c + 1 for c in cols)

        pltpu.async_copy(outb.at[r], out_hbm.at[l, :, pl.ds(b0, BBLK)], osem)

    fire(0, 0)
    fire(1, 1)

    @pl.loop(0, L - NBUF, step=NBUF)
    def _chunk(l0):
        for r in range(NBUF):
            consume(r, l0 + r)
        for r in range(NBUF):
            drain_store(r, l0 + r)
            fire(r, l0 + NBUF + r)

    for r in range(NBUF):
        consume(r, L - NBUF + r)
    for r in range(NBUF):
        drain_store(r, L - NBUF + r)


_mesh = plsc.VectorSubcoreMesh(core_axis_name="c", subcore_axis_name="s")

_gather_call = pl.kernel(
    _body,
    out_type=jax.ShapeDtypeStruct((L, DIM, B), jnp.float32),
    mesh=_mesh,
    scratch_types=[
        pltpu.VMEM((L, BBLK), jnp.int32),
        pltpu.VMEM((NBUF, BBLK), jnp.int32),
        pltpu.VMEM((NBUF, BBLK, 128), jnp.float32),
        pltpu.VMEM((NBUF, DIM, BBLK), jnp.float32),
        pltpu.SemaphoreType.DMA,
        pltpu.SemaphoreType.DMA,
    ],
    compiler_params=pltpu.CompilerParams(needs_layout_passes=False),
)


@jax.jit
def _run(idxT, w2):
    return _gather_call(idxT, w2)


def kernel(gene_indices, weight):
    idxT = jnp.transpose(jnp.asarray(gene_indices, jnp.int32))
    w2 = jnp.reshape(weight, (PAIRS, 128))
    out_t = _run(idxT, w2)
    return jnp.transpose(out_t, (2, 0, 1))


# speedup vs baseline: 1.1298x; 1.1204x over previous
import jax
import jax.numpy as jnp
from jax import lax
from jax.experimental import pallas as pl
from jax.experimental.pallas import tpu as pltpu
from jax.experimental.pallas import tpu_sc as plsc

B = 4096
L = 200
DIM = 64
PAIRS = 500000

NC = 2
NS = 16
NW = NC * NS
BBLK = B // NW
NG = BBLK // 16
NBUF = 2


def _body(idxT_hbm, w2_hbm, out_hbm, idx_v, jbuf, pairs_v, outb, gsem, osem):
    cid = lax.axis_index("c")
    sid = lax.axis_index("s")
    wid = sid * NC + cid
    b0 = wid * BBLK

    pltpu.sync_copy(idxT_hbm.at[:, pl.ds(b0, BBLK)], idx_v)

    lanes = lax.iota(jnp.int32, 16)
    rows = [g * 16 + lanes for g in range(NG)]

    def fire(r, l):
        for g in range(NG):
            v = idx_v[l, pl.ds(g * 16, 16)]
            jbuf[r, pl.ds(g * 16, 16)] = v >> 1
        pltpu.async_copy(w2_hbm.at[jbuf.at[r]], pairs_v.at[r], gsem)

    def drain_store(r, l):
        pltpu.make_async_copy(
            outb.at[r, :, pl.ds(0, BBLK)], out_hbm.at[l, :, pl.ds(b0, BBLK)], osem
        ).wait()

    def consume(r, l):
        pltpu.make_async_copy(w2_hbm.at[jbuf.at[r]], pairs_v.at[r], gsem).wait()
        @pl.loop(0, NG)
        def _gloop(g):
            c0 = g * 16
            offs = (idx_v[l, pl.ds(c0, 16)] & 1) * 64
            for i in range(16):
                colbase = offs[i] + lanes
                cvec = jnp.full((16,), 0, jnp.int32) + (c0 + i)
                for f0 in range(0, DIM, 16):
                    vec = plsc.load_gather(pairs_v.at[r], [cvec, colbase + f0])
                    plsc.store_scatter(outb.at[r], [f0 + lanes, cvec], vec)

        pltpu.async_copy(
            outb.at[r, :, pl.ds(0, BBLK)], out_hbm.at[l, :, pl.ds(b0, BBLK)], osem
        )

    fire(0, 0)
    fire(1, 1)

    @pl.loop(0, L - NBUF, step=NBUF)
    def _chunk(l0):
        for r in range(NBUF):
            consume(r, l0 + r)
        for r in range(NBUF):
            drain_store(r, l0 + r)
            fire(r, l0 + NBUF + r)

    for r in range(NBUF):
        consume(r, L - NBUF + r)
    for r in range(NBUF):
        drain_store(r, L - NBUF + r)


_mesh = plsc.VectorSubcoreMesh(core_axis_name="c", subcore_axis_name="s")

_gather_call = pl.kernel(
    _body,
    out_type=jax.ShapeDtypeStruct((L, DIM, B), jnp.float32),
    mesh=_mesh,
    scratch_types=[
        pltpu.VMEM((L, BBLK), jnp.int32),
        pltpu.VMEM((NBUF, BBLK), jnp.int32),
        pltpu.VMEM((NBUF, BBLK, 128), jnp.float32),
        pltpu.VMEM((NBUF, DIM, BBLK + 1), jnp.float32),
        pltpu.SemaphoreType.DMA,
        pltpu.SemaphoreType.DMA,
    ],
    compiler_params=pltpu.CompilerParams(needs_layout_passes=False),
)


@jax.jit
def _run(idxT, w2):
    return _gather_call(idxT, w2)


def kernel(gene_indices, weight):
    idxT = jnp.transpose(jnp.asarray(gene_indices, jnp.int32))
    w2 = jnp.reshape(weight, (PAIRS, 128))
    out_t = _run(idxT, w2)
    return jnp.transpose(out_t, (2, 0, 1))


# speedup vs baseline: 1.6077x; 1.4230x over previous
import jax
import jax.numpy as jnp
from jax import lax
from jax.experimental import pallas as pl
from jax.experimental.pallas import tpu as pltpu
from jax.experimental.pallas import tpu_sc as plsc

B = 4096
L = 200
DIM = 64

NC = 2
NS = 16
NW = NC * NS
BPW = B // NW
NBUF = 4
S1 = 104
S2 = L - S1


def _body(idx_hbm, w_hbm, out_hbm, idx_v, rows_v, gsem, osem):
    cid = lax.axis_index("c")
    sid = lax.axis_index("s")
    wid = sid * NC + cid
    b0 = wid * BPW

    pltpu.sync_copy(idx_hbm.at[pl.ds(b0, BPW)], idx_v)

    def fire(r, k):
        pltpu.async_copy(
            w_hbm.at[idx_v.at[k, pl.ds(0, S1)]],
            rows_v.at[r, pl.ds(0, S1)],
            gsem,
        )
        pltpu.async_copy(
            w_hbm.at[idx_v.at[k, pl.ds(S1, S2)]],
            rows_v.at[r, pl.ds(S1, S2)],
            gsem,
        )

    def consume(r, k):
        pltpu.make_async_copy(
            w_hbm.at[idx_v.at[k, pl.ds(0, S1)]],
            rows_v.at[r, pl.ds(0, S1)],
            gsem,
        ).wait()
        pltpu.make_async_copy(
            w_hbm.at[idx_v.at[k, pl.ds(S1, S2)]],
            rows_v.at[r, pl.ds(S1, S2)],
            gsem,
        ).wait()
        pltpu.async_copy(rows_v.at[r], out_hbm.at[b0 + k], osem)

    def drain(r, k):
        pltpu.make_async_copy(rows_v.at[r], out_hbm.at[b0 + k], osem).wait()

    for r in range(NBUF):
        fire(r, r)

    @pl.loop(0, BPW - NBUF, step=NBUF)
    def _loop(k0):
        for r in range(NBUF):
            consume(r, k0 + r)
        for r in range(NBUF):
            drain(r, k0 + r)
            fire(r, k0 + NBUF + r)

    for r in range(NBUF):
        consume(r, BPW - NBUF + r)
    for r in range(NBUF):
        drain(r, BPW - NBUF + r)


_mesh = plsc.VectorSubcoreMesh(core_axis_name="c", subcore_axis_name="s")

_gather_call = pl.kernel(
    _body,
    out_type=jax.ShapeDtypeStruct((B, L, DIM), jnp.float32),
    mesh=_mesh,
    scratch_types=[
        pltpu.VMEM((BPW, L), jnp.int32),
        pltpu.VMEM((NBUF, L, DIM), jnp.float32),
        pltpu.SemaphoreType.DMA,
        pltpu.SemaphoreType.DMA,
    ],
    compiler_params=pltpu.CompilerParams(use_tc_tiling_on_sc=False),
)


@jax.jit
def _run(idx, weight):
    return _gather_call(idx, weight)


def kernel(gene_indices, weight):
    return _run(jnp.asarray(gene_indices, jnp.int32), weight)


# speedup vs baseline: 1.6732x; 1.0408x over previous
import jax
import jax.numpy as jnp
from jax import lax
from jax.experimental import pallas as pl
from jax.experimental.pallas import tpu as pltpu
from jax.experimental.pallas import tpu_sc as plsc

B = 4096
L = 200
DIM = 64
PAIRS = 500000

NC = 2
NS = 16
NW = NC * NS
BBLK = B // NW
NG = BBLK // 16
PITCH = BBLK + 8
NBUF = 2


def _body(idxT_hbm, w2_hbm, out_hbm, idx_v, jbuf, pairs_v, outb, gsem, osem):
    cid = lax.axis_index("c")
    sid = lax.axis_index("s")
    wid = sid * NC + cid
    b0 = wid * BBLK

    pltpu.sync_copy(idxT_hbm.at[:, pl.ds(b0, BBLK)], idx_v)

    lanes = lax.iota(jnp.int32, 16)
    rot = [(lanes + d) & 15 for d in range(16)]

    def fire(r, l):
        for g in range(NG):
            v = idx_v[l, pl.ds(g * 16, 16)]
            jbuf[r, pl.ds(g * 16, 16)] = v >> 1
        pltpu.async_copy(w2_hbm.at[jbuf.at[r]], pairs_v.at[r], gsem)

    def drain_store(r, l):
        pltpu.make_async_copy(
            outb.at[r, :, pl.ds(0, BBLK)], out_hbm.at[l, :, pl.ds(b0, BBLK)], osem
        ).wait()

    def consume(r, l):
        pltpu.make_async_copy(w2_hbm.at[jbuf.at[r]], pairs_v.at[r], gsem).wait()
        @pl.loop(0, NG)
        def _gloop(g):
            c0 = g * 16
            offs = (idx_v[l, pl.ds(c0, 16)] & 1) * 64
            crow = c0 + lanes
            for f0 in range(0, DIM, 16):
                for d in range(16):
                    fvec = rot[d] + f0
                    vec = plsc.load_gather(pairs_v.at[r], [crow, offs + fvec])
                    plsc.store_scatter(outb.at[r], [fvec, crow], vec)

        pltpu.async_copy(
            outb.at[r, :, pl.ds(0, BBLK)], out_hbm.at[l, :, pl.ds(b0, BBLK)], osem
        )

    fire(0, 0)
    fire(1, 1)

    @pl.loop(0, L - NBUF, step=NBUF)
    def _chunk(l0):
        for r in range(NBUF):
            consume(r, l0 + r)
        for r in range(NBUF):
            drain_store(r, l0 + r)
            fire(r, l0 + NBUF + r)

    for r in range(NBUF):
        consume(r, L - NBUF + r)
    for r in range(NBUF):
        drain_store(r, L - NBUF + r)


_mesh = plsc.VectorSubcoreMesh(core_axis_name="c", subcore_axis_name="s")

_gather_call = pl.kernel(
    _body,
    out_type=jax.ShapeDtypeStruct((L, DIM, B), jnp.float32),
    mesh=_mesh,
    scratch_types=[
        pltpu.VMEM((L, BBLK), jnp.int32),
        pltpu.VMEM((NBUF, BBLK), jnp.int32),
        pltpu.VMEM((NBUF, BBLK, 128), jnp.float32),
        pltpu.VMEM((NBUF, DIM, PITCH), jnp.float32),
        pltpu.SemaphoreType.DMA,
        pltpu.SemaphoreType.DMA,
    ],
    compiler_params=pltpu.CompilerParams(needs_layout_passes=False),
)


@jax.jit
def _run(idxT, w2):
    return _gather_call(idxT, w2)


def kernel(gene_indices, weight):
    idxT = jnp.transpose(jnp.asarray(gene_indices, jnp.int32))
    w2 = jnp.reshape(weight, (PAIRS, 128))
    out_t = _run(idxT, w2)
    return jnp.transpose(out_t, (2, 0, 1))


# speedup vs baseline: 1.6780x; 1.0028x over previous
import jax
import jax.numpy as jnp
from jax import lax
from jax.experimental import pallas as pl
from jax.experimental.pallas import tpu as pltpu
from jax.experimental.pallas import tpu_sc as plsc

B = 4096
L = 200
DIM = 64
PAIRS = 500000

NC = 2
NS = 16
NW = NC * NS
BBLK = B // NW
NG = BBLK // 16
PITCH = BBLK + 8
NBUF = 2


def _body(idxT_hbm, w2_hbm, out_hbm, idx_v, jbuf, pairs_v, outb, gsem, osem):
    cid = lax.axis_index("c")
    sid = lax.axis_index("s")
    wid = sid * NC + cid
    b0 = wid * BBLK

    pltpu.sync_copy(idxT_hbm.at[:, pl.ds(b0, BBLK)], idx_v)

    lanes = lax.iota(jnp.int32, 16)
    rot_f0 = [
        [((lanes + d) & 15) + f0 for d in range(16)]
        for f0 in range(0, DIM, 16)
    ]

    def fire(r, l):
        for g in range(NG):
            v = idx_v[l, pl.ds(g * 16, 16)]
            jbuf[r, pl.ds(g * 16, 16)] = v >> 1
        pltpu.async_copy(w2_hbm.at[jbuf.at[r]], pairs_v.at[r], gsem)

    def drain_store(r, l):
        pltpu.make_async_copy(
            outb.at[r, :, pl.ds(0, BBLK)], out_hbm.at[l, :, pl.ds(b0, BBLK)], osem
        ).wait()

    def consume(r, l):
        pltpu.make_async_copy(w2_hbm.at[jbuf.at[r]], pairs_v.at[r], gsem).wait()
        @pl.loop(0, NG)
        def _gloop(g):
            c0 = g * 16
            offs = (idx_v[l, pl.ds(c0, 16)] & 1) * 64
            crow = c0 + lanes
            for fi in range(DIM // 16):
                for d in range(16):
                    fvec = rot_f0[fi][d]
                    vec = plsc.load_gather(pairs_v.at[r], [crow, offs + fvec])
                    plsc.store_scatter(outb.at[r], [fvec, crow], vec)

        pltpu.async_copy(
            outb.at[r, :, pl.ds(0, BBLK)], out_hbm.at[l, :, pl.ds(b0, BBLK)], osem
        )

    fire(0, 0)
    fire(1, 1)

    @pl.loop(0, L - NBUF, step=NBUF)
    def _chunk(l0):
        for r in range(NBUF):
            consume(r, l0 + r)
        for r in range(NBUF):
            drain_store(r, l0 + r)
            fire(r, l0 + NBUF + r)

    for r in range(NBUF):
        consume(r, L - NBUF + r)
    for r in range(NBUF):
        drain_store(r, L - NBUF + r)


_mesh = plsc.VectorSubcoreMesh(core_axis_name="c", subcore_axis_name="s")

_gather_call = pl.kernel(
    _body,
    out_type=jax.ShapeDtypeStruct((L, DIM, B), jnp.float32),
    mesh=_mesh,
    scratch_types=[
        pltpu.VMEM((L, BBLK), jnp.int32),
        pltpu.VMEM((NBUF, BBLK), jnp.int32),
        pltpu.VMEM((NBUF, BBLK, 128), jnp.float32),
        pltpu.VMEM((NBUF, DIM, PITCH), jnp.float32),
        pltpu.SemaphoreType.DMA,
        pltpu.SemaphoreType.DMA,
    ],
    compiler_params=pltpu.CompilerParams(needs_layout_passes=False),
)


@jax.jit
def _run(idxT, w2):
    return _gather_call(idxT, w2)


def kernel(gene_indices, weight):
    idxT = jnp.transpose(jnp.asarray(gene_indices, jnp.int32))
    w2 = jnp.reshape(weight, (PAIRS, 128))
    out_t = _run(idxT, w2)
    return jnp.transpose(out_t, (2, 0, 1))
